# baseline (speedup 1.0000x reference)
import functools

import jax
import jax.numpy as jnp
from jax import lax
from jax.experimental import pallas as pl
from jax.experimental.pallas import tpu as pltpu
from jax.experimental.pallas import tpu_sc as plsc

_B, _C, _HW = 32, 384, 1024
_CHW = _C * _HW
_CPC = 48
_CH = _CPC * _HW
_N = _C // _CPC
_K = 2


def _sc_body(x_hbm, mid_hbm, out_hbm, sp, mask_v, idx_v, zeros_v,
             is0, is1, os0, os1):
    isems = (is0, is1)
    osems = (os0, os1)

    cid = lax.axis_index("c")
    sid = lax.axis_index("s")
    wid = sid * 2 + cid

    pltpu.sync_copy(mid_hbm, mask_v)
    wid_vec = jnp.zeros((16,), jnp.int32) + wid
    mid_vec = plsc.load_gather(mask_v, [wid_vec])

    for i in range(8):
        zeros_v[pl.ds(i * 16, 16)] = jnp.zeros((16,), jnp.float32)

    lane = lax.iota(jnp.int32, 16)
    base = wid * _CHW
    for v in range(24):
        vec = lane * _HW + mid_vec + (v * 16 * _HW + base)
        idx_v[v // 8, pl.ds((v % 8) * 16, 16)] = vec

    def in_copy(k):
        return pltpu.make_async_copy(
            x_hbm.at[pl.ds(base + k * _CH, _CH)],
            sp.at[sid, k % _K], isems[k % _K])

    def out_copy(k):
        return pltpu.make_async_copy(
            sp.at[sid, k % _K],
            out_hbm.at[pl.ds(base + k * _CH, _CH)], osems[k % _K])

    in_copy(0).start()
    in_copy(1).start()
    for k in range(_N):
        in_copy(k).wait()
        out_copy(k).start()
        if k + _K < _N:
            out_copy(k).wait()
            in_copy(k + _K).start()
    for k in range(_N - _K, _N):
        out_copy(k).wait()

    for v in range(3):
        pltpu.sync_copy(zeros_v, out_hbm.at[idx_v.at[v]])


def kernel(fea, mask_id):
    b, c, h, w = fea.shape
    x = fea.reshape(b * c * h * w)
    mesh = plsc.VectorSubcoreMesh(core_axis_name="c", subcore_axis_name="s")
    run = functools.partial(
        pl.kernel,
        mesh=mesh,
        compiler_params=pltpu.CompilerParams(needs_layout_passes=False),
        out_type=jax.ShapeDtypeStruct((b * c * h * w,), jnp.float32),
        scratch_types=(
            [pltpu.VMEM_SHARED((16, _K, _CH), jnp.float32)]
            + [pltpu.VMEM((_B,), jnp.int32)]
            + [pltpu.VMEM((3, 128), jnp.int32)]
            + [pltpu.VMEM((128,), jnp.float32)]
            + [pltpu.SemaphoreType.DMA for _ in range(4)]
        ),
    )(_sc_body)
    out = run(x, mask_id)
    return out.reshape(b, c, h, w)


# speedup vs baseline: 3.6049x; 3.6049x over previous
import jax
import jax.numpy as jnp
from jax.experimental import pallas as pl
from jax.experimental.pallas import tpu as pltpu

_P = 8
_NPH = 4


def _body(x_ref, mid_ref, o_ref, ibuf0, ibuf1, obuf0, obuf1, isem, osem):
    hw = x_ref.shape[-1]
    pos = jax.lax.broadcasted_iota(jnp.int32, (1, hw), 1)
    ibufs = (ibuf0, ibuf1)
    obufs = (obuf0, obuf1)

    def stage_in(ph):
        for j in range(_P):
            pltpu.make_async_copy(
                x_ref.at[ph * _P + j], ibufs[ph % 2].at[j], isem
            ).start(priority=j % 2)

    def wait_in(ph):
        pltpu.make_async_copy(
            x_ref.at[pl.ds(ph * _P, _P)], ibufs[ph % 2], isem).wait()

    def compute(ph):
        for j in range(_P):
            mid = mid_ref[ph * _P + j]
            obufs[ph % 2][j] = jnp.where(pos == mid, 0.0, ibufs[ph % 2][j])

    def stage_out(ph):
        for j in range(_P):
            pltpu.make_async_copy(
                obufs[ph % 2].at[j], o_ref.at[ph * _P + j], osem
            ).start(priority=j % 2)

    def wait_out(ph):
        pltpu.make_async_copy(
            obufs[ph % 2], o_ref.at[pl.ds(ph * _P, _P)], osem).wait()

    stage_in(0)
    for ph in range(_NPH):
        wait_in(ph)
        if ph >= 2:
            wait_out(ph - 2)
        compute(ph)
        stage_out(ph)
        if ph + 1 < _NPH:
            stage_in(ph + 1)
    wait_out(_NPH - 2)
    wait_out(_NPH - 1)


def kernel(fea, mask_id):
    b, c, h, w = fea.shape
    hw = h * w
    x = fea.reshape(b, c, hw)
    out = pl.pallas_call(
        _body,
        grid=(1,),
        in_specs=[
            pl.BlockSpec(memory_space=pl.ANY),
            pl.BlockSpec(memory_space=pltpu.SMEM),
        ],
        out_specs=pl.BlockSpec(memory_space=pl.ANY),
        out_shape=jax.ShapeDtypeStruct((b, c, hw), jnp.float32),
        scratch_shapes=[
            pltpu.VMEM((_P, c, hw), jnp.float32),
            pltpu.VMEM((_P, c, hw), jnp.float32),
            pltpu.VMEM((_P, c, hw), jnp.float32),
            pltpu.VMEM((_P, c, hw), jnp.float32),
            pltpu.SemaphoreType.DMA,
            pltpu.SemaphoreType.DMA,
        ],
    )(x, mask_id)
    return out.reshape(b, c, h, w)


# speedup vs baseline: 3.7855x; 1.0501x over previous
import jax
import jax.numpy as jnp
from jax.experimental import pallas as pl
from jax.experimental.pallas import tpu as pltpu

_K = 8
_NT = 2


def _body(x_ref, mid_ref, o_ref, ibuf, obuf, isem, osem):
    n = x_ref.shape[0]
    hw = x_ref.shape[-1]
    pos = jax.lax.broadcasted_iota(jnp.int32, (1, hw), 1)

    def in_copy(k):
        return pltpu.make_async_copy(x_ref.at[k], ibuf.at[k % _K], isem.at[k % _K])

    def out_copy(k):
        return pltpu.make_async_copy(obuf.at[k % _K], o_ref.at[k], osem.at[k % _K])

    for k in range(_K):
        in_copy(k).start(priority=k % _NT)
    for k in range(n):
        in_copy(k).wait()
        if k >= _K:
            out_copy(k - _K).wait()
        mid = mid_ref[k]
        obuf[k % _K] = jnp.where(pos == mid, 0.0, ibuf[k % _K])
        out_copy(k).start(priority=k % _NT)
        if k + _K < n:
            in_copy(k + _K).start(priority=(k + _K) % _NT)
    for k in range(n - _K, n):
        out_copy(k).wait()


def kernel(fea, mask_id):
    b, c, h, w = fea.shape
    hw = h * w
    x = fea.reshape(b, c, hw)
    out = pl.pallas_call(
        _body,
        grid=(1,),
        in_specs=[
            pl.BlockSpec(memory_space=pl.ANY),
            pl.BlockSpec(memory_space=pltpu.SMEM),
        ],
        out_specs=pl.BlockSpec(memory_space=pl.ANY),
        out_shape=jax.ShapeDtypeStruct((b, c, hw), jnp.float32),
        scratch_shapes=[
            pltpu.VMEM((_K, c, hw), jnp.float32),
            pltpu.VMEM((_K, c, hw), jnp.float32),
            pltpu.SemaphoreType.DMA((_K,)),
            pltpu.SemaphoreType.DMA((_K,)),
        ],
    )(x, mask_id)
    return out.reshape(b, c, h, w)
